# baseline (speedup 1.0000x reference)
import jax
import jax.numpy as jnp
from jax import lax
from jax.experimental import pallas as pl
from jax.experimental.pallas import tpu as pltpu
from jax.experimental.pallas import tpu_sc as plsc

_TOK = 2048
_NC, _NS, _LN = 2, 16, 16
_NW = _NC * _NS
_CH = 128


def _tc_body(x_ref, sx_ref, esq_ref, iotak_ref, e_ref,
             idx_ref, counts_ref, scal_ref, acc_ref):
    i = pl.program_id(0)
    n_steps = pl.num_programs(0)
    K = e_ref.shape[1]
    xc = jnp.clip(x_ref[...], -10.0, 10.0)
    mm2 = jnp.dot(xc * -2.0, e_ref[...], preferred_element_type=jnp.float32)
    d = (sx_ref[...] + esq_ref[...]) + mm2
    m = jnp.min(d, axis=1, keepdims=True)
    oh = jnp.where(d == m, 1.0, 0.0)
    idxf = jnp.min(iotak_ref[...] - jnp.float32(K) * oh, axis=1, keepdims=True)
    idx_ref[...] = idxf.astype(jnp.int32)
    ones = jnp.ones((1, oh.shape[0]), jnp.float32)
    cnt = jnp.dot(ones, oh, preferred_element_type=jnp.float32)
    lsum = jnp.sum(m, axis=0, keepdims=True)

    @pl.when(i == 0)
    def _():
        counts_ref[...] = cnt
        acc_ref[...] = lsum

    @pl.when(i != 0)
    def _():
        counts_ref[...] += cnt
        acc_ref[...] += lsum

    @pl.when(i == n_steps - 1)
    def _():
        n_tok = jnp.float32(_TOK) * n_steps
        mean_sq = acc_ref[...] / (n_tok * x_ref.shape[1])
        loss = mean_sq + 0.25 * mean_sq
        avg = counts_ref[...] / n_tok
        ent = jnp.sum(avg * jnp.log(avg + 1e-10), axis=1, keepdims=True)
        perp = jnp.exp(-ent)
        scal_ref[:, 0:1] = loss
        scal_ref[:, 1:2] = perp


def _sc_body(emb_hbm, idx_hbm, qst_hbm, idx_v, rows_a, rows_b, rows_c,
             sem_a, sem_b, sem_c, wsem_a, wsem_b, wsem_c):
    bpw = idx_v.shape[0]
    nch = bpw // _CH
    nb = 3
    wid = lax.axis_index("s") * _NC + lax.axis_index("c")
    base = wid * bpw
    pltpu.sync_copy(idx_hbm.at[pl.ds(base, bpw)], idx_v)

    rows = (rows_a, rows_b, rows_c)
    sems = (sem_a, sem_b, sem_c)
    wsems = (wsem_a, wsem_b, wsem_c)

    def gather(c):
        return pltpu.async_copy(
            emb_hbm.at[idx_v.at[pl.ds(c * _CH, _CH)]],
            rows[c % nb], sems[c % nb])

    cps = [None] * nch
    wps = [None] * nch
    waited = [False] * nch
    for p in range(min(nb, nch)):
        cps[p] = gather(p)
    for c in range(nch):
        cps[c].wait()
        wps[c] = pltpu.async_copy(
            rows[c % nb], qst_hbm.at[pl.ds(base + c * _CH, _CH)],
            wsems[c % nb])
        if c + nb < nch:
            wps[c].wait()
            waited[c] = True
            cps[c + nb] = gather(c + nb)
    for c in range(nch):
        if not waited[c]:
            wps[c].wait()


def kernel(inputs, embedding):
    B, D, H, W = inputs.shape
    K = embedding.shape[1]
    N = B * H * W

    x_perm = jnp.transpose(inputs.astype(jnp.float32), (0, 2, 3, 1))
    flat = x_perm.reshape(N, D)
    flat_c = jnp.clip(flat, -10.0, 10.0)
    s_x = jnp.sum(flat_c ** 2, axis=1, keepdims=True)
    e_sq = jnp.sum(embedding ** 2, axis=0, keepdims=True)
    iota_k = (jnp.arange(K, dtype=jnp.float32) + K).reshape(1, K)
    e_t_q = embedding.T.astype(jnp.bfloat16)
    e_t_packed = lax.bitcast_convert_type(
        e_t_q.reshape(K, D // 2, 2), jnp.int32)

    grid = N // _TOK
    idx, counts, scal, _ = pl.pallas_call(
        _tc_body,
        grid=(grid,),
        in_specs=[
            pl.BlockSpec((_TOK, D), lambda i: (i, 0)),
            pl.BlockSpec((_TOK, 1), lambda i: (i, 0)),
            pl.BlockSpec((1, K), lambda i: (0, 0)),
            pl.BlockSpec((1, K), lambda i: (0, 0)),
            pl.BlockSpec((D, K), lambda i: (0, 0)),
        ],
        out_specs=[
            pl.BlockSpec((_TOK, 1), lambda i: (i, 0)),
            pl.BlockSpec((1, K), lambda i: (0, 0)),
            pl.BlockSpec((1, 2), lambda i: (0, 0)),
            pl.BlockSpec((1, 1), lambda i: (0, 0)),
        ],
        out_shape=[
            jax.ShapeDtypeStruct((N, 1), jnp.int32),
            jax.ShapeDtypeStruct((1, K), jnp.float32),
            jax.ShapeDtypeStruct((1, 2), jnp.float32),
            jax.ShapeDtypeStruct((1, 1), jnp.float32),
        ],
    )(flat, s_x, e_sq, iota_k, embedding)

    idx_flat = idx.reshape(N)
    bpw = N // _NW
    Dp = D // 2
    mesh = plsc.VectorSubcoreMesh(core_axis_name="c", subcore_axis_name="s")
    qst = pl.kernel(
        _sc_body,
        mesh=mesh,
        out_type=jax.ShapeDtypeStruct((N, Dp), jnp.int32),
        scratch_types=[
            pltpu.VMEM((bpw,), jnp.int32),
            pltpu.VMEM((_CH, Dp), jnp.int32),
            pltpu.VMEM((_CH, Dp), jnp.int32),
            pltpu.VMEM((_CH, Dp), jnp.int32),
            pltpu.SemaphoreType.DMA,
            pltpu.SemaphoreType.DMA,
            pltpu.SemaphoreType.DMA,
            pltpu.SemaphoreType.DMA,
            pltpu.SemaphoreType.DMA,
            pltpu.SemaphoreType.DMA,
        ],
    )(e_t_packed, idx_flat)

    qbf = lax.bitcast_convert_type(qst, jnp.bfloat16).reshape(N, D)
    qst32 = qbf.astype(jnp.float32)
    quantized_st = jnp.transpose(qst32.reshape(B, H, W, D), (0, 3, 1, 2))
    quantized_st = quantized_st.astype(inputs.dtype)
    loss = scal[0, 0]
    perplexity = scal[0, 1]
    return (quantized_st, loss, perplexity, idx_flat)
